# baseline (speedup 1.0000x reference)
import functools

import jax
import jax.numpy as jnp
from jax import lax
from jax.experimental import pallas as pl
from jax.experimental.pallas import tpu as pltpu
from jax.experimental.pallas import tpu_sc as plsc

D = 128
NC = 2
NS = 16
NW = NC * NS
CHUNK = 128
BR = 512


NBUF = 2


@functools.lru_cache(maxsize=None)
def _make_segsum(n_pad, nchunk):
    assert nchunk % 2 == 0 and nchunk >= 4
    rows_per_tile = n_pad // NS
    mesh = plsc.VectorSubcoreMesh(core_axis_name="c", subcore_axis_name="s")

    @functools.partial(
        pl.kernel,
        out_type=jax.ShapeDtypeStruct((NC, n_pad, D), jnp.float32),
        mesh=mesh,
        scratch_types=[
            pltpu.VMEM_SHARED((n_pad, D), jnp.float32),
            [pltpu.VMEM((CHUNK,), jnp.int32) for _ in range(NBUF)],
            [pltpu.VMEM((CHUNK,), jnp.int32) for _ in range(NBUF)],
            [pltpu.VMEM((CHUNK, D), jnp.float32) for _ in range(NBUF)],
            [pltpu.SemaphoreType.DMA for _ in range(NBUF)],
            [pltpu.SemaphoreType.DMA for _ in range(NBUF)],
        ],
    )
    def segsum(src_hbm, dst_hbm, table_hbm, zeros_hbm, out_hbm,
               acc, sidx, didx, bufs, isems, gsems):
        c = lax.axis_index("c")
        s = lax.axis_index("s")
        r0 = pl.multiple_of(s * rows_per_tile, 8)
        pltpu.sync_copy(zeros_hbm.at[pl.ds(r0, rows_per_tile)],
                        acc.at[pl.ds(r0, rows_per_tile)])
        plsc.subcore_barrier()

        w = s * NC + c
        src_my = src_hbm.at[w]
        dst_my = dst_hbm.at[w]

        def fire_idx(j, b):
            pltpu.async_copy(src_my.at[j], sidx[b], isems[b])
            pltpu.async_copy(dst_my.at[j], didx[b], isems[b])

        def wait_idx(j, b):
            pltpu.make_async_copy(src_my.at[j], sidx[b], isems[b]).wait()
            pltpu.make_async_copy(dst_my.at[j], didx[b], isems[b]).wait()

        def fire_gather(b):
            pltpu.async_copy(table_hbm.at[sidx[b]], bufs[b], gsems[b])

        def step(j, b, bn, fire_next_gather, fire_next_idx):
            if fire_next_gather:
                wait_idx(j + 1, bn)
                fire_gather(bn)
            pltpu.make_async_copy(
                table_hbm.at[sidx[b]], bufs[b], gsems[b]).wait()
            pltpu.sync_copy(bufs[b], acc.at[didx[b]], add=True)
            if fire_next_idx:
                fire_idx(j + 2, b)

        fire_idx(0, 0)
        fire_idx(1, 1)
        wait_idx(0, 0)
        fire_gather(0)

        @pl.loop(0, nchunk - 2, step=2)
        def _(g):
            step(g, 0, 1, True, True)
            step(g + 1, 1, 0, True, True)

        step(nchunk - 2, 0, 1, True, False)
        step(nchunk - 1, 1, 0, False, False)

        plsc.subcore_barrier()
        pltpu.sync_copy(acc.at[pl.ds(r0, rows_per_tile)],
                        out_hbm.at[c].at[pl.ds(r0, rows_per_tile)])

    return segsum


def _tc1_body(p_ref, x_ref, wr_ref, wo_ref, b_ref, h_ref):
    agg = p_ref[0] + p_ref[1]
    h = (jnp.dot(agg, wr_ref[...], preferred_element_type=jnp.float32)
         + jnp.dot(x_ref[...], wo_ref[...], preferred_element_type=jnp.float32)
         + b_ref[...])
    h_ref[...] = jnp.maximum(h, 0.0)


def _tc2_body(p_ref, h_ref, wr_ref, wo_ref, b_ref, o_ref):
    agg = p_ref[0] + p_ref[1]
    o = (jnp.dot(agg, wr_ref[...], preferred_element_type=jnp.float32)
         + jnp.dot(h_ref[...], wo_ref[...], preferred_element_type=jnp.float32)
         + b_ref[...])
    o = o - jnp.max(o, axis=1, keepdims=True)
    o_ref[...] = o - jnp.log(jnp.sum(jnp.exp(o), axis=1, keepdims=True))


def _tc_layer(body, partials, dense_in, w_rel, w_root, b, n_pad):
    grid = (n_pad // BR,)
    return pl.pallas_call(
        body,
        grid=grid,
        in_specs=[
            pl.BlockSpec((NC, BR, D), lambda i: (0, i, 0)),
            pl.BlockSpec((BR, D), lambda i: (i, 0)),
            pl.BlockSpec((D, D), lambda i: (0, 0)),
            pl.BlockSpec((D, D), lambda i: (0, 0)),
            pl.BlockSpec((1, D), lambda i: (0, 0)),
        ],
        out_specs=pl.BlockSpec((BR, D), lambda i: (i, 0)),
        out_shape=jax.ShapeDtypeStruct((n_pad, D), jnp.float32),
    )(partials, dense_in, w_rel, w_root, b.reshape(1, D))


def kernel(x, edge_index, W1_rel, W1_root, b1, W2_rel, W2_root, b2):
    n = x.shape[0]
    e = edge_index.shape[1]
    n_pad = -(-(n + 1) // BR) * BR
    nchunk = -(-e // (NW * CHUNK * NBUF)) * NBUF
    e_pad = nchunk * CHUNK * NW

    src = jnp.concatenate(
        [edge_index[0], jnp.zeros((e_pad - e,), jnp.int32)]
    ).reshape(NW, nchunk, CHUNK)
    dst = jnp.concatenate(
        [edge_index[1], jnp.full((e_pad - e,), n, jnp.int32)]
    ).reshape(NW, nchunk, CHUNK)
    zeros = jnp.zeros((n_pad, D), jnp.float32)
    x_pad = jnp.concatenate([x, jnp.zeros((n_pad - n, D), jnp.float32)], axis=0)

    segsum = _make_segsum(n_pad, nchunk)
    p1 = segsum(src, dst, x_pad, zeros)
    h = _tc_layer(_tc1_body, p1, x_pad, W1_rel, W1_root, b1, n_pad)
    p2 = segsum(src, dst, h, zeros)
    out = _tc_layer(_tc2_body, p2, h, W2_rel, W2_root, b2, n_pad)
    return out[:n]


# speedup vs baseline: 3.3405x; 3.3405x over previous
import functools

import jax
import jax.numpy as jnp
from jax import lax
from jax.experimental import pallas as pl
from jax.experimental.pallas import tpu as pltpu
from jax.experimental.pallas import tpu_sc as plsc

D = 128
NC = 2
NS = 16
NW = NC * NS
CHUNK = 128
BR = 512


NBUF = 2


@functools.lru_cache(maxsize=None)
def _make_segsum(n_pad, nchunk):
    assert nchunk % 2 == 0 and nchunk >= 4
    rows_per_tile = n_pad // NS
    mesh = plsc.VectorSubcoreMesh(core_axis_name="c", subcore_axis_name="s")

    @functools.partial(
        pl.kernel,
        out_type=jax.ShapeDtypeStruct((NC, n_pad, D), jnp.float32),
        mesh=mesh,
        scratch_types=[
            pltpu.VMEM_SHARED((n_pad, D), jnp.float32),
            [pltpu.VMEM((CHUNK,), jnp.int32) for _ in range(NBUF)],
            [pltpu.VMEM((CHUNK,), jnp.int32) for _ in range(NBUF)],
            [pltpu.VMEM((CHUNK, D), jnp.float32) for _ in range(NBUF)],
            [pltpu.SemaphoreType.DMA for _ in range(NBUF)],
            [pltpu.SemaphoreType.DMA for _ in range(NBUF)],
        ],
    )
    def segsum(src_hbm, dst_hbm, table_hbm, zeros_hbm, out_hbm,
               acc, sidx, didx, bufs, isems, gsems):
        c = lax.axis_index("c")
        s = lax.axis_index("s")
        r0 = pl.multiple_of(s * rows_per_tile, 8)
        pltpu.sync_copy(zeros_hbm.at[pl.ds(r0, rows_per_tile)],
                        acc.at[pl.ds(r0, rows_per_tile)])
        plsc.subcore_barrier()

        w = s * NC + c
        src_my = src_hbm.at[w]
        dst_my = dst_hbm.at[w]

        def fire_idx(j, b):
            pltpu.async_copy(src_my.at[j], sidx[b], isems[b])
            pltpu.async_copy(dst_my.at[j], didx[b], isems[b])

        def wait_idx(j, b):
            pltpu.make_async_copy(src_my.at[j], sidx[b], isems[b]).wait()
            pltpu.make_async_copy(dst_my.at[j], didx[b], isems[b]).wait()

        def fire_gather(b):
            pltpu.async_copy(table_hbm.at[sidx[b]], bufs[b], gsems[b])

        def step(j, b, bn, fire_next_gather, fire_next_idx):
            if fire_next_gather:
                wait_idx(j + 1, bn)
                fire_gather(bn)
            pltpu.make_async_copy(
                table_hbm.at[sidx[b]], bufs[b], gsems[b]).wait()
            pltpu.sync_copy(bufs[b], acc.at[didx[b]], add=True)
            if fire_next_idx:
                fire_idx(j + 2, b)

        fire_idx(0, 0)
        fire_idx(1, 1)
        wait_idx(0, 0)
        fire_gather(0)

        @pl.loop(0, nchunk - 2, step=2)
        def _(g):
            step(g, 0, 1, True, True)
            step(g + 1, 1, 0, True, True)

        step(nchunk - 2, 0, 1, True, False)
        step(nchunk - 1, 1, 0, False, False)

        plsc.subcore_barrier()
        pltpu.sync_copy(acc.at[pl.ds(r0, rows_per_tile)],
                        out_hbm.at[c].at[pl.ds(r0, rows_per_tile)])

    return segsum


def _tc1_body(p_ref, x_ref, wr_ref, wo_ref, b_ref, h_ref):
    agg = p_ref[0] + p_ref[1]
    h = (jnp.dot(agg, wr_ref[...], preferred_element_type=jnp.float32)
         + jnp.dot(x_ref[...], wo_ref[...], preferred_element_type=jnp.float32)
         + b_ref[...])
    h_ref[...] = jnp.maximum(h, 0.0)


def _tc2_body(p_ref, h_ref, wr_ref, wo_ref, b_ref, o_ref):
    agg = p_ref[0] + p_ref[1]
    o = (jnp.dot(agg, wr_ref[...], preferred_element_type=jnp.float32)
         + jnp.dot(h_ref[...], wo_ref[...], preferred_element_type=jnp.float32)
         + b_ref[...])
    o = o - jnp.max(o, axis=1, keepdims=True)
    o_ref[...] = o - jnp.log(jnp.sum(jnp.exp(o), axis=1, keepdims=True))


def _tc_layer(body, partials, dense_in, w_rel, w_root, b, n_pad):
    grid = (n_pad // BR,)
    return pl.pallas_call(
        body,
        grid=grid,
        in_specs=[
            pl.BlockSpec((NC, BR, D), lambda i: (0, i, 0)),
            pl.BlockSpec((BR, D), lambda i: (i, 0)),
            pl.BlockSpec((D, D), lambda i: (0, 0)),
            pl.BlockSpec((D, D), lambda i: (0, 0)),
            pl.BlockSpec((1, D), lambda i: (0, 0)),
        ],
        out_specs=pl.BlockSpec((BR, D), lambda i: (i, 0)),
        out_shape=jax.ShapeDtypeStruct((n_pad, D), jnp.float32),
    )(partials, dense_in, w_rel, w_root, b.reshape(1, D))


def kernel(x, edge_index, W1_rel, W1_root, b1, W2_rel, W2_root, b2):
    n = x.shape[0]
    e = edge_index.shape[1]
    n_pad = -(-(n + 1) // BR) * BR
    nchunk = -(-e // (NW * CHUNK * NBUF)) * NBUF
    e_pad = nchunk * CHUNK * NW

    pad_ar = jnp.arange(e_pad - e, dtype=jnp.int32)
    src = jnp.concatenate(
        [edge_index[0], pad_ar % n]
    ).reshape(NW, nchunk, CHUNK)
    dst = jnp.concatenate(
        [edge_index[1], n + pad_ar % (n_pad - n)]
    ).reshape(NW, nchunk, CHUNK)
    zeros = jnp.zeros((n_pad, D), jnp.float32)
    x_pad = jnp.concatenate([x, jnp.zeros((n_pad - n, D), jnp.float32)], axis=0)

    segsum = _make_segsum(n_pad, nchunk)
    p1 = segsum(src, dst, x_pad, zeros)
    h = _tc_layer(_tc1_body, p1, x_pad, W1_rel, W1_root, b1, n_pad)
    p2 = segsum(src, dst, h, zeros)
    out = _tc_layer(_tc2_body, p2, h, W2_rel, W2_root, b2, n_pad)
    return out[:n]
